# baseline (speedup 1.0000x reference)
import functools

import jax
import jax.numpy as jnp
from jax import lax
from jax.experimental import pallas as pl
from jax.experimental.pallas import tpu as pltpu
from jax.experimental.pallas import tpu_sc as plsc

NC = 2
NS = 16
L = 16

CH = 4096
NHALF = 2


@functools.partial(jax.jit, static_argnums=(4, 5, 6))
def _sc_scatter(x_t0, x_t1, idx_t, src_t, m, d, b):
  nw = NC * NS
  cols_per_w = d // nw
  nchunk = b // CH
  m2 = m // NHALF
  nslab = cols_per_w * NHALF

  mesh = plsc.VectorSubcoreMesh(
      core_axis_name="c", subcore_axis_name="s", num_cores=NC,
      num_subcores=NS)

  def body(x0, x1, idx_ref, src_ref, o0, o1, cb0, cb1, ivb0, ivb1, svb0,
           svb1, lsem0, lsem1, ssem0, ssem1, isem0, isem1):
    w = lax.axis_index("s") * NC + lax.axis_index("c")
    xs = [x0, x1]
    os = [o0, o1]
    cb = [cb0, cb1]
    lsem = [lsem0, lsem1]
    ssem = [ssem0, ssem1]
    ivb = [ivb0, ivb1]
    svb = [svb0, svb1]
    isem = [isem0, isem1]

    def col_of(s):
      return w * cols_per_w + (s // NHALF)

    def half_of(s):
      return s % NHALF

    def stage_chunk(c, h):
      pltpu.async_copy(idx_ref.at[c, pl.ds(h * CH, CH)], ivb[h % 2],
                       isem[h % 2])
      pltpu.async_copy(src_ref.at[c, pl.ds(h * CH, CH)], svb[h % 2],
                       isem[h % 2])

    def wait_chunk(c, h):
      pltpu.make_async_copy(
          idx_ref.at[c, pl.ds(h * CH, CH)], ivb[h % 2], isem[h % 2]).wait()
      pltpu.make_async_copy(
          src_ref.at[c, pl.ds(h * CH, CH)], svb[h % 2], isem[h % 2]).wait()

    pltpu.async_copy(xs[half_of(0)].at[col_of(0)], cb[0], lsem[0])
    stage_chunk(col_of(0), 0)

    for s in range(nslab):
      nb = s % 2
      col = col_of(s)
      half = half_of(s)

      if s + 1 < nslab:
        if s >= 1:
          pltpu.make_async_copy(
              cb[1 - nb], os[half_of(s - 1)].at[col_of(s - 1)],
              ssem[1 - nb]).wait()
        pltpu.async_copy(
            xs[half_of(s + 1)].at[col_of(s + 1)], cb[1 - nb], lsem[1 - nb])

      pltpu.make_async_copy(
          xs[half].at[col], cb[nb], lsem[nb]).wait()

      for h in range(nchunk):
        if h + 1 < nchunk:
          stage_chunk(col, h + 1)
        elif s + 1 < nslab:
          stage_chunk(col_of(s + 1), 0)
        wait_chunk(col, h)

        def v1(k, _, hb=h % 2, half=half, nb=nb):
          iv = ivb[hb][pl.ds(k * L, L)]
          sv = svb[hb][pl.ds(k * L, L)]
          if half == 0:
            plsc.store_scatter(cb[nb], [iv], sv, mask=iv < m2)
          else:
            plsc.store_scatter(cb[nb], [iv - m2], sv, mask=iv >= m2)
          return 0
        lax.fori_loop(0, CH // L, v1, 0)

      pltpu.async_copy(cb[nb], os[half].at[col], ssem[nb])

    pltpu.make_async_copy(
        cb[(nslab - 1) % 2], os[half_of(nslab - 1)].at[col_of(nslab - 1)],
        ssem[(nslab - 1) % 2]).wait()
    if nslab >= 2:
      pltpu.make_async_copy(
          cb[nslab % 2], os[half_of(nslab - 2)].at[col_of(nslab - 2)],
          ssem[nslab % 2]).wait()

  fn = pl.kernel(
      body,
      out_type=(jax.ShapeDtypeStruct((d, m2), jnp.float32),
                jax.ShapeDtypeStruct((d, m2), jnp.float32)),
      mesh=mesh,
      compiler_params=pltpu.CompilerParams(needs_layout_passes=False),
      scratch_types=[
          pltpu.VMEM((m2,), jnp.float32),
          pltpu.VMEM((m2,), jnp.float32),
          pltpu.VMEM((CH,), jnp.int32),
          pltpu.VMEM((CH,), jnp.int32),
          pltpu.VMEM((CH,), jnp.float32),
          pltpu.VMEM((CH,), jnp.float32),
          pltpu.SemaphoreType.DMA,
          pltpu.SemaphoreType.DMA,
          pltpu.SemaphoreType.DMA,
          pltpu.SemaphoreType.DMA,
          pltpu.SemaphoreType.DMA,
          pltpu.SemaphoreType.DMA,
      ],
      name="scatter_overwrite_sc",
  )
  return fn(x_t0, x_t1, idx_t, src_t)


def kernel(x, dim, index, src):
  m, d = x.shape
  b = src.shape[0]
  m2 = m // NHALF
  rows = (index + dim).astype(jnp.int32)
  out_t0, out_t1 = _sc_scatter(
      x[:m2].T, x[m2:].T, rows.T, src.T, m, d, b)
  return jnp.concatenate([out_t0.T, out_t1.T], axis=0)


# speedup vs baseline: 1.4665x; 1.4665x over previous
import functools

import jax
import jax.numpy as jnp
from jax import lax
from jax.experimental import pallas as pl
from jax.experimental.pallas import tpu as pltpu
from jax.experimental.pallas import tpu_sc as plsc

NC = 2
NS = 16
L = 16

CH = 4096


@functools.partial(jax.jit, static_argnums=(3, 4, 5))
def _sc_scatter(x_t, idx_t, src_t, m, d, b):
  nw = NC * NS
  cols_per_w = d // nw
  nchunk = b // CH

  mesh = plsc.VectorSubcoreMesh(
      core_axis_name="c", subcore_axis_name="s", num_cores=NC,
      num_subcores=NS)

  def body(x_ref, idx_ref, src_ref, out_ref, colbuf, ivb0, ivb1, svb0, svb1,
           csem, osem, isem0, isem1):
    w = lax.axis_index("s") * NC + lax.axis_index("c")
    ivb = [ivb0, ivb1]
    svb = [svb0, svb1]
    isem = [isem0, isem1]

    for lc in range(cols_per_w):
      col = w * cols_per_w + lc

      cdesc = pltpu.async_copy(x_ref.at[col], colbuf, csem)
      pltpu.async_copy(idx_ref.at[col, pl.ds(0, CH)], ivb[0], isem[0])
      pltpu.async_copy(src_ref.at[col, pl.ds(0, CH)], svb[0], isem[0])
      cdesc.wait()

      for h in range(nchunk):
        nxt = h + 1
        if nxt < nchunk:
          pltpu.async_copy(
              idx_ref.at[col, pl.ds(nxt * CH, CH)], ivb[nxt % 2],
              isem[nxt % 2])
          pltpu.async_copy(
              src_ref.at[col, pl.ds(nxt * CH, CH)], svb[nxt % 2],
              isem[nxt % 2])
        pltpu.make_async_copy(
            idx_ref.at[col, pl.ds(h * CH, CH)], ivb[h % 2], isem[h % 2]
        ).wait()
        pltpu.make_async_copy(
            src_ref.at[col, pl.ds(h * CH, CH)], svb[h % 2], isem[h % 2]
        ).wait()

        def v1(k, _, h=h):
          iv = ivb[h % 2][pl.ds(k * L, L)]
          sv = svb[h % 2][pl.ds(k * L, L)]
          plsc.store_scatter(colbuf, [iv], sv)
          return 0
        lax.fori_loop(0, CH // L, v1, 0)

      odesc = pltpu.async_copy(colbuf, out_ref.at[col], osem)
      odesc.wait()

  fn = pl.kernel(
      body,
      out_type=jax.ShapeDtypeStruct((d, m), jnp.float32),
      mesh=mesh,
      compiler_params=pltpu.CompilerParams(needs_layout_passes=False),
      scratch_types=[
          pltpu.VMEM((m,), jnp.float32),
          pltpu.VMEM((CH,), jnp.int32),
          pltpu.VMEM((CH,), jnp.int32),
          pltpu.VMEM((CH,), jnp.float32),
          pltpu.VMEM((CH,), jnp.float32),
          pltpu.SemaphoreType.DMA,
          pltpu.SemaphoreType.DMA,
          pltpu.SemaphoreType.DMA,
          pltpu.SemaphoreType.DMA,
      ],
      name="scatter_overwrite_sc",
  )
  return fn(x_t, idx_t, src_t)


def kernel(x, dim, index, src):
  m, d = x.shape
  b = src.shape[0]
  rows = (index + dim).astype(jnp.int32)
  out_t = _sc_scatter(x.T, rows.T, src.T, m, d, b)
  return out_t.T
